# baseline (speedup 1.0000x reference)
import dataclasses

import jax
import jax.numpy as jnp
from jax import lax
from jax.experimental import pallas as pl
from jax.experimental.pallas import tpu as pltpu
from jax.experimental.pallas import tpu_sc as plsc

EMBED_DIM = 128
GAZ_DIM = 64
OUT_DIM = EMBED_DIM + GAZ_DIM
WINDOW = 128
LANES = 16


def _embed_concat(sentence_data, gazetteers_data, word_table):
    num_tokens = sentence_data.shape[0]
    nw = num_tokens // WINDOW
    idx2d = sentence_data.reshape(nw, WINDOW)
    gaz_t3 = gazetteers_data.T.reshape(GAZ_DIM // 8, 8, num_tokens)
    mesh = plsc.VectorSubcoreMesh(core_axis_name="core",
                                  subcore_axis_name="subcore")

    cp = pltpu.CompilerParams()
    if "needs_layout_passes" in pltpu.CompilerParams.__dataclass_fields__:
        cp = dataclasses.replace(cp, needs_layout_passes=False)

    @pl.kernel(
        out_type=jax.ShapeDtypeStruct((OUT_DIM // 8, nw, 8, WINDOW),
                                      jnp.float32),
        mesh=mesh,
        compiler_params=cp,
        scratch_types=[pltpu.VMEM((WINDOW, EMBED_DIM), jnp.float32),
                       pltpu.SemaphoreType.DMA,
                       pltpu.SemaphoreType.DMA],
    )
    def kern(idx_hbm, gazt_hbm, table_hbm, out_hbm, scr, gsem, zsem):
        def body(indices, i_vmem, o_vmem):
            (w,) = indices
            zcp = pltpu.async_copy(
                gazt_hbm.at[:, :, pl.ds(w * WINDOW, WINDOW)],
                o_vmem.at[pl.ds(EMBED_DIM // 8, GAZ_DIM // 8), 0], zsem)
            gcp = pltpu.async_copy(table_hbm.at[i_vmem.at[0]], scr, gsem)
            gcp.wait()

            tok = lax.iota(jnp.int32, LANES)
            zero = jnp.zeros((LANES,), jnp.int32)
            base = [(tok + k * LANES) * EMBED_DIM
                    for k in range(WINDOW // LANES)]
            for jb in range(EMBED_DIM // 8):
                for jr in range(8):
                    f = jb * 8 + jr
                    for k in range(WINDOW // LANES):
                        vals = plsc.load_gather(scr, [zero, base[k] + f])
                        o_vmem[jb, 0, jr, pl.ds(k * LANES, LANES)] = vals

            zcp.wait()

        pltpu.emit_pipeline(
            body,
            grid=(nw,),
            in_specs=[
                pl.BlockSpec((1, WINDOW), lambda i: (i, 0)),
            ],
            out_specs=[
                pl.BlockSpec((OUT_DIM // 8, 1, 8, WINDOW),
                             lambda i: (0, i, 0, 0)),
            ],
            core_axis_name=("core", "subcore"),
            dimension_semantics=(pltpu.PARALLEL,),
            _explicit_indices=True,
        )(idx_hbm, out_hbm)

    out_tiled = kern(idx2d, gaz_t3, word_table)
    return out_tiled.transpose(1, 3, 0, 2).reshape(num_tokens, OUT_DIM)


def kernel(sentence_data, batch_sizes, gazetteers_data, word_table):
    out = _embed_concat(sentence_data, gazetteers_data, word_table)
    return out, batch_sizes


# speedup vs baseline: 2.3264x; 2.3264x over previous
import dataclasses

import jax
import jax.numpy as jnp
from jax import lax
from jax.experimental import pallas as pl
from jax.experimental.pallas import tpu as pltpu
from jax.experimental.pallas import tpu_sc as plsc

EMBED_DIM = 128
GAZ_DIM = 64
OUT_DIM = EMBED_DIM + GAZ_DIM
WINDOW = 128
LANES = 16


def _embed_concat(sentence_data, gazetteers_data, word_table):
    num_tokens = sentence_data.shape[0]
    nw = num_tokens // WINDOW
    idx2d = sentence_data.reshape(nw, WINDOW)
    gaz_t3 = gazetteers_data.T.reshape(GAZ_DIM // 8, 8, num_tokens)
    mesh = plsc.VectorSubcoreMesh(core_axis_name="core",
                                  subcore_axis_name="subcore")

    cp = pltpu.CompilerParams()
    if "needs_layout_passes" in pltpu.CompilerParams.__dataclass_fields__:
        cp = dataclasses.replace(cp, needs_layout_passes=False)

    @pl.kernel(
        out_type=jax.ShapeDtypeStruct((OUT_DIM // 8, nw, 8, WINDOW),
                                      jnp.float32),
        mesh=mesh,
        compiler_params=cp,
        scratch_types=[pltpu.VMEM((WINDOW, EMBED_DIM), jnp.float32),
                       pltpu.SemaphoreType.DMA,
                       pltpu.SemaphoreType.DMA],
    )
    def kern(idx_hbm, gazt_hbm, table_hbm, out_hbm, scr, gsem, zsem):
        def body(indices, i_vmem, o_vmem):
            (w,) = indices
            zcp = pltpu.async_copy(
                gazt_hbm.at[:, :, pl.ds(w * WINDOW, WINDOW)],
                o_vmem.at[pl.ds(EMBED_DIM // 8, GAZ_DIM // 8), 0], zsem)
            gcp = pltpu.async_copy(table_hbm.at[i_vmem.at[0]], scr, gsem)
            gcp.wait()

            tok = lax.iota(jnp.int32, LANES)
            zero = jnp.zeros((LANES,), jnp.int32)
            rpat = [tok * EMBED_DIM + ((tok + d) & (LANES - 1))
                    for d in range(LANES)]
            wpat = [((tok + d) & (LANES - 1)) * WINDOW + tok
                    for d in range(LANES)]
            @pl.loop(0, WINDOW // LANES)
            def _(k):
                t0 = k * LANES

                @pl.loop(0, EMBED_DIM // LANES)
                def _(g):
                    f0 = g * LANES
                    for d in range(LANES):
                        vals = plsc.load_gather(
                            scr, [zero, rpat[d] + (t0 * EMBED_DIM + f0)])
                        plsc.store_scatter(
                            o_vmem, [zero, zero, zero,
                                     wpat[d] + (f0 * WINDOW + t0)], vals)

            zcp.wait()

        pltpu.emit_pipeline(
            body,
            grid=(nw,),
            in_specs=[
                pl.BlockSpec((1, WINDOW), lambda i: (i, 0)),
            ],
            out_specs=[
                pl.BlockSpec((OUT_DIM // 8, 1, 8, WINDOW),
                             lambda i: (0, i, 0, 0)),
            ],
            core_axis_name=("core", "subcore"),
            dimension_semantics=(pltpu.PARALLEL,),
            _explicit_indices=True,
        )(idx_hbm, out_hbm)

    out_tiled = kern(idx2d, gaz_t3, word_table)
    return out_tiled.transpose(1, 3, 0, 2).reshape(num_tokens, OUT_DIM)


def kernel(sentence_data, batch_sizes, gazetteers_data, word_table):
    out = _embed_concat(sentence_data, gazetteers_data, word_table)
    return out, batch_sizes


# speedup vs baseline: 3.2738x; 1.4072x over previous
import dataclasses

import jax
import jax.numpy as jnp
from jax import lax
from jax.experimental import pallas as pl
from jax.experimental.pallas import tpu as pltpu
from jax.experimental.pallas import tpu_sc as plsc

EMBED_DIM = 128
GAZ_DIM = 64
OUT_DIM = EMBED_DIM + GAZ_DIM
WINDOW = 128
LANES = 16
NWORK = 32


def _embed_concat(sentence_data, gazetteers_data, word_table):
    num_tokens = sentence_data.shape[0]
    nw = num_tokens // WINDOW
    pw = nw // NWORK
    idx3d = sentence_data.reshape(NWORK, pw, WINDOW)
    gaz_t4 = gazetteers_data.T.reshape(GAZ_DIM // 8, 1, 8, num_tokens)
    mesh = plsc.VectorSubcoreMesh(core_axis_name="core",
                                  subcore_axis_name="subcore")

    cp = pltpu.CompilerParams()
    if "needs_layout_passes" in pltpu.CompilerParams.__dataclass_fields__:
        cp = dataclasses.replace(cp, needs_layout_passes=False)

    @pl.kernel(
        out_type=jax.ShapeDtypeStruct((OUT_DIM // 8, nw, 8, WINDOW),
                                      jnp.float32),
        mesh=mesh,
        compiler_params=cp,
        scratch_types=[
            pltpu.VMEM((pw, WINDOW), jnp.int32),
            pltpu.VMEM((WINDOW, EMBED_DIM), jnp.float32),
            pltpu.VMEM((WINDOW, EMBED_DIM), jnp.float32),
            pltpu.VMEM((OUT_DIM // 8, 1, 8, WINDOW), jnp.float32),
            pltpu.VMEM((OUT_DIM // 8, 1, 8, WINDOW), jnp.float32),
            pltpu.SemaphoreType.DMA,
            pltpu.SemaphoreType.DMA,
            pltpu.SemaphoreType.DMA,
            pltpu.SemaphoreType.DMA,
            pltpu.SemaphoreType.DMA,
            pltpu.SemaphoreType.DMA,
            pltpu.SemaphoreType.DMA,
        ],
    )
    def kern(idx_hbm, gazt_hbm, table_hbm, out_hbm,
             idx_all, scr_a, scr_b, ob_a, ob_b,
             isem, gsem_a, gsem_b, zsem_a, zsem_b, osem_a, osem_b):
        wid = lax.axis_index("subcore") * 2 + lax.axis_index("core")
        base = wid * pw

        pltpu.async_copy(idx_hbm.at[wid], idx_all, isem).wait()
        pltpu.async_copy(table_hbm.at[idx_all.at[0]], scr_a, gsem_a)
        pltpu.async_copy(table_hbm.at[idx_all.at[1]], scr_b, gsem_b)

        tok = lax.iota(jnp.int32, LANES)
        zero = jnp.zeros((LANES,), jnp.int32)
        rpat = [tok * EMBED_DIM + ((tok + d) & (LANES - 1))
                for d in range(LANES)]
        wpat = [((tok + d) & (LANES - 1)) * WINDOW + tok
                for d in range(LANES)]

        def out_dst(j):
            return out_hbm.at[pl.ds(0, OUT_DIM // 8), pl.ds(base + j, 1)]

        def stage(j, scr, ob, gsem, zsem, osem):
            @pl.when(j >= 2)
            def _():
                pltpu.make_async_copy(ob, out_dst(0), osem).wait()

            pltpu.async_copy(
                gazt_hbm.at[:, :, :, pl.ds((base + j) * WINDOW, WINDOW)],
                ob.at[pl.ds(EMBED_DIM // 8, GAZ_DIM // 8)], zsem)

            pltpu.make_async_copy(table_hbm.at[pl.ds(0, WINDOW)], scr,
                                  gsem).wait()

            @pl.loop(0, WINDOW // LANES)
            def _(k):
                t0 = k * LANES

                @pl.loop(0, EMBED_DIM // LANES)
                def _(g):
                    f0 = g * LANES
                    for d in range(LANES):
                        vals = plsc.load_gather(
                            scr, [zero, rpat[d] + (t0 * EMBED_DIM + f0)])
                        plsc.store_scatter(
                            ob, [zero, zero, zero,
                                 wpat[d] + (f0 * WINDOW + t0)], vals)

            @pl.when(j + 2 < pw)
            def _():
                pltpu.async_copy(table_hbm.at[idx_all.at[j + 2]], scr, gsem)

            pltpu.make_async_copy(
                gazt_hbm.at[:, :, :, pl.ds(0, WINDOW)],
                ob.at[pl.ds(EMBED_DIM // 8, GAZ_DIM // 8)], zsem).wait()
            pltpu.async_copy(ob, out_dst(j), osem)

        @pl.loop(0, pw, step=2)
        def _(j):
            stage(j, scr_a, ob_a, gsem_a, zsem_a, osem_a)
            stage(j + 1, scr_b, ob_b, gsem_b, zsem_b, osem_b)

        pltpu.make_async_copy(ob_a, out_dst(0), osem_a).wait()
        pltpu.make_async_copy(ob_b, out_dst(0), osem_b).wait()

    out_tiled = kern(idx3d, gaz_t4, word_table)
    return out_tiled.transpose(1, 3, 0, 2).reshape(num_tokens, OUT_DIM)


def kernel(sentence_data, batch_sizes, gazetteers_data, word_table):
    out = _embed_concat(sentence_data, gazetteers_data, word_table)
    return out, batch_sizes
